# baseline (speedup 1.0000x reference)
import functools

import jax
import jax.numpy as jnp
from jax import lax
from jax.experimental import pallas as pl
from jax.experimental.pallas import tpu as pltpu
from jax.experimental.pallas import tpu_sc as plsc

N = 10000
E = 320000
D = 128
H = 64
G = 16
OUT = 2

NC = 2
NS = 16
LN = 16
NW = NC * NS

CH = 128
NCHUNK = 81
EP = NW * NCHUNK * CH
NP = 10240
RPT = NP // NS

@functools.cache
def _sc_mesh():
    return plsc.VectorSubcoreMesh(core_axis_name="c", subcore_axis_name="s",
                                  num_cores=NC, num_subcores=NS)


def _zero_rows(rows):
    z16 = jnp.zeros((LN,), jnp.float32)

    def body(i, carry):
        r = i // (H // LN)
        q = i % (H // LN)
        rows[r, pl.ds(q * LN, LN)] = z16
        return carry

    lax.fori_loop(0, CH * (H // LN), body, 0, unroll=8)


def _deg_body(fidx_hbm, fw_hbm, aidx_hbm, aw_hbm, out_hbm,
              accf, acca, idx_v, w_v, zrow):
    cid = lax.axis_index("c")
    sid = lax.axis_index("s")
    wid = sid * NC + cid

    z16 = jnp.zeros((LN,), jnp.float32)

    def zb(i, c):
        zrow[pl.ds(i * LN, LN)] = z16
        return c

    lax.fori_loop(0, RPT // LN, zb, 0, unroll=8)
    pltpu.sync_copy(zrow, accf.at[pl.ds(sid * RPT, RPT)])
    pltpu.sync_copy(zrow, acca.at[pl.ds(sid * RPT, RPT)])
    plsc.subcore_barrier()

    def run(idx_hbm, w_hbm, acc):
        pltpu.sync_copy(idx_hbm.at[wid], idx_v)
        pltpu.sync_copy(w_hbm.at[wid], w_v)

        def body(j, c):
            pltpu.sync_copy(w_v.at[j], acc.at[idx_v.at[j]], add=True)
            return c

        lax.fori_loop(0, NCHUNK, body, 0)

    run(fidx_hbm, fw_hbm, accf)
    run(aidx_hbm, aw_hbm, acca)
    plsc.subcore_barrier()
    pltpu.sync_copy(accf.at[pl.ds(sid * RPT, RPT)],
                    out_hbm.at[cid, 0, pl.ds(sid * RPT, RPT)])
    pltpu.sync_copy(acca.at[pl.ds(sid * RPT, RPT)],
                    out_hbm.at[cid, 1, pl.ds(sid * RPT, RPT)])


@functools.cache
def _deg_kernel():
    return pl.kernel(
        _deg_body,
        out_type=jax.ShapeDtypeStruct((NC, 2, NP), jnp.float32),
        mesh=_sc_mesh(),
        compiler_params=pltpu.CompilerParams(use_tc_tiling_on_sc=False),
        scratch_types=[
            pltpu.VMEM_SHARED((NP,), jnp.float32),
            pltpu.VMEM_SHARED((NP,), jnp.float32),
            pltpu.VMEM((NCHUNK, CH), jnp.int32),
            pltpu.VMEM((NCHUNK, CH), jnp.float32),
            pltpu.VMEM((RPT,), jnp.float32),
        ],
    )


def _scale_rows(rows, w_v, j):
    for gi in range(CH // LN):
        wv = w_v[j, pl.ds(gi * LN, LN)]
        for ee in range(LN):
            wb = jnp.full((LN,), wv[ee], jnp.float32)
            e = gi * LN + ee
            for q in range(H // LN):
                rows[e, pl.ds(q * LN, LN)] = rows[e, pl.ds(q * LN, LN)] * wb


def _spmm_body(g_hbm, sidx_hbm, didx_hbm, w_hbm, out_hbm,
               acc, sidx_v, didx_v, w_v, rows0, rows1, rows2, zbuf,
               gsem0, gsem1, gsem2, ssem0, ssem1, ssem2):
    cid = lax.axis_index("c")
    sid = lax.axis_index("s")
    wid = sid * NC + cid
    rows = (rows0, rows1, rows2)
    gsem = (gsem0, gsem1, gsem2)
    ssem = (ssem0, ssem1, ssem2)

    pltpu.sync_copy(sidx_hbm.at[wid], sidx_v)
    pltpu.sync_copy(didx_hbm.at[wid], didx_v)
    pltpu.sync_copy(w_hbm.at[wid], w_v)

    pltpu.async_copy(g_hbm.at[sidx_v.at[0]], rows0, gsem0)
    pltpu.async_copy(g_hbm.at[sidx_v.at[1]], rows1, gsem1)

    _zero_rows(zbuf)
    for k in range(RPT // CH):
        pltpu.sync_copy(zbuf, acc.at[pl.ds(sid * RPT + k * CH, CH)])
    plsc.subcore_barrier()

    def body(i, c):
        j0 = 3 * i
        for b in range(3):
            j = j0 + b
            pltpu.make_async_copy(g_hbm.at[sidx_v.at[j]], rows[b],
                                  gsem[b]).wait()
            _scale_rows(rows[b], w_v, j)
            pltpu.async_copy(rows[b], acc.at[didx_v.at[j]], ssem[b], add=True)
            bp = (b + 2) % 3

            @pl.when(j >= 1)
            def _():
                pltpu.make_async_copy(rows[bp], acc.at[didx_v.at[j - 1]],
                                      ssem[bp]).wait()

            @pl.when(j + 2 < NCHUNK)
            def _():
                pltpu.async_copy(g_hbm.at[sidx_v.at[j + 2]], rows[bp],
                                 gsem[bp])
        return c

    lax.fori_loop(0, NCHUNK // 3, body, 0)

    pltpu.make_async_copy(rows[(NCHUNK - 1) % 3],
                          acc.at[didx_v.at[NCHUNK - 1]],
                          ssem[(NCHUNK - 1) % 3]).wait()

    plsc.subcore_barrier()
    pltpu.sync_copy(acc.at[pl.ds(sid * RPT, RPT)],
                    out_hbm.at[cid, pl.ds(sid * RPT, RPT)])


@functools.cache
def _spmm_kernel():
    return pl.kernel(
        _spmm_body,
        out_type=jax.ShapeDtypeStruct((NC, NP, H), jnp.float32),
        mesh=_sc_mesh(),
        compiler_params=pltpu.CompilerParams(use_tc_tiling_on_sc=False),
        scratch_types=[
            pltpu.VMEM_SHARED((NP, H), jnp.float32),
            pltpu.VMEM((NCHUNK, CH), jnp.int32),
            pltpu.VMEM((NCHUNK, CH), jnp.int32),
            pltpu.VMEM((NCHUNK, CH), jnp.float32),
            pltpu.VMEM((CH, H), jnp.float32),
            pltpu.VMEM((CH, H), jnp.float32),
            pltpu.VMEM((CH, H), jnp.float32),
            pltpu.VMEM((CH, H), jnp.float32),
            pltpu.SemaphoreType.DMA,
            pltpu.SemaphoreType.DMA,
            pltpu.SemaphoreType.DMA,
            pltpu.SemaphoreType.DMA,
            pltpu.SemaphoreType.DMA,
            pltpu.SemaphoreType.DMA,
        ],
    )


def _bn_cols(y, g, be):
    m = jnp.mean(y, axis=0, keepdims=True)
    v = jnp.mean(y * y, axis=0, keepdims=True) - m * m
    return (y - m) * lax.rsqrt(v + 1e-5) * g[None, :] + be[None, :]


def _tc_prep_body(x_ref, w_ref, degp_ref, hw_ref, g_ref, dinvf_ref, dinva_ref):
    degf = degp_ref[0, 0, :N] + degp_ref[1, 0, :N] + 1.0
    dega = degp_ref[0, 1, :N] + degp_ref[1, 1, :N] + 1.0
    dinvf = jnp.where(degf > 0, lax.rsqrt(degf), 0.0)[:, None]
    dinva = jnp.where(dega > 0, lax.rsqrt(dega), 0.0)[:, None]
    hw = jnp.dot(x_ref[...], w_ref[...], preferred_element_type=jnp.float32)
    hw_ref[...] = hw
    g_ref[...] = dinvf * hw
    dinvf_ref[...] = dinvf
    dinva_ref[...] = dinva


_tc_prep = pl.pallas_call(
    _tc_prep_body,
    out_shape=[
        jax.ShapeDtypeStruct((N, H), jnp.float32),
        jax.ShapeDtypeStruct((N, H), jnp.float32),
        jax.ShapeDtypeStruct((N, 1), jnp.float32),
        jax.ShapeDtypeStruct((N, 1), jnp.float32),
    ],
)


def _post(aggp_ref, hw_ref, dinv_ref, b_ref, g_ref, be_ref):
    dinv = dinv_ref[...]
    agg = aggp_ref[0, :N, :] + aggp_ref[1, :N, :]
    y = dinv * agg + (dinv * dinv) * hw_ref[...] + b_ref[...][None, :]
    return _bn_cols(y, g_ref[...], be_ref[...])


def _tc_mid0_body(aggp_ref, hw_ref, dinvf_ref, b_ref, g_ref, be_ref, w1_ref,
                  hw1_ref, g1_ref):
    h0 = jax.nn.relu(_post(aggp_ref, hw_ref, dinvf_ref, b_ref, g_ref, be_ref))
    hw1 = jnp.dot(h0, w1_ref[...], preferred_element_type=jnp.float32)
    hw1_ref[...] = hw1
    g1_ref[...] = dinvf_ref[...] * hw1


_tc_mid0 = pl.pallas_call(
    _tc_mid0_body,
    out_shape=[
        jax.ShapeDtypeStruct((N, H), jnp.float32),
        jax.ShapeDtypeStruct((N, H), jnp.float32),
    ],
)


def _tc_mid1_body(aggp_ref, hw_ref, dinvf_ref, dinva_ref, b_ref, g_ref, be_ref,
                  wa_ref, wf_ref, hwa_ref, ga_ref, hwf_ref, gf_ref):
    h1 = _post(aggp_ref, hw_ref, dinvf_ref, b_ref, g_ref, be_ref)
    hwa = jnp.dot(h1, wa_ref[...], preferred_element_type=jnp.float32)
    hwa_ref[...] = hwa
    ga_ref[...] = dinva_ref[...] * hwa
    hwf = jnp.dot(h1, wf_ref[...], preferred_element_type=jnp.float32)
    hwf_ref[...] = hwf
    gf_ref[...] = dinvf_ref[...] * hwf


_tc_mid1 = pl.pallas_call(
    _tc_mid1_body,
    out_shape=[
        jax.ShapeDtypeStruct((N, H), jnp.float32),
        jax.ShapeDtypeStruct((N, H), jnp.float32),
        jax.ShapeDtypeStruct((N, H), jnp.float32),
        jax.ShapeDtypeStruct((N, H), jnp.float32),
    ],
)


def _tc_pool_body(aggp_ref, hw_ref, dinv_ref, b_ref, g_ref, be_ref, batch_ref,
                  pool_ref):
    h = _post(aggp_ref, hw_ref, dinv_ref, b_ref, g_ref, be_ref)
    oh = (batch_ref[...] == lax.broadcasted_iota(jnp.int32, (1, G), 1)
          ).astype(jnp.float32)
    cnt = jnp.maximum(jnp.sum(oh, axis=0), 1.0)[:, None]
    dn = (((0,), (0,)), ((), ()))
    pool_ref[...] = lax.dot_general(oh, h, dn,
                                    preferred_element_type=jnp.float32) / cnt


_tc_pool = pl.pallas_call(
    _tc_pool_body,
    out_shape=jax.ShapeDtypeStruct((G, H), jnp.float32),
)


def _tc_head_body(pa_ref, pf_ref, wc1_ref, bc1_ref, wc2_ref, bc2_ref, out_ref):
    combined = jnp.concatenate([pa_ref[...], pf_ref[...]], axis=1)
    z = jax.nn.relu(jnp.dot(combined, wc1_ref[...],
                            preferred_element_type=jnp.float32)
                    + bc1_ref[...][None, :])
    out_ref[...] = (jnp.dot(z, wc2_ref[...], preferred_element_type=jnp.float32)
                    + bc2_ref[...][None, :])


_tc_head = pl.pallas_call(
    _tc_head_body,
    out_shape=jax.ShapeDtypeStruct((G, OUT), jnp.float32),
)


def _prep_edges(edge_index, edge_attr):
    src = edge_index[0].astype(jnp.int32)
    dst = edge_index[1].astype(jnp.int32)
    w = edge_attr[:, 0].astype(jnp.float32)
    pad = EP - E
    pidx = (jnp.arange(pad, dtype=jnp.int32) * 37) % N
    src = jnp.concatenate([src, pidx]).reshape(NW, NCHUNK, CH)
    dst = jnp.concatenate([dst, pidx]).reshape(NW, NCHUNK, CH)
    w = jnp.concatenate([w, jnp.zeros((pad,), jnp.float32)]
                        ).reshape(NW, NCHUNK, CH)
    return src, dst, w


def kernel(x, func_edge_index, func_edge_attr, anat_edge_index, anat_edge_attr,
           batch, W_s0, b_s0, g_s0, be_s0, W_s1, b_s1, g_s1, be_s1,
           W_a, b_a, g_a, be_a, W_f, b_f, g_f, be_f, Wc1, bc1, Wc2, bc2):
    fs, fd, fw = _prep_edges(func_edge_index, func_edge_attr)
    asrc, adst, aw = _prep_edges(anat_edge_index, anat_edge_attr)
    batch2d = batch.astype(jnp.int32)[:, None]

    degp = _deg_kernel()(fd, fw, adst, aw)
    hw0, g0, dinvf, dinva = _tc_prep(x, W_s0, degp)

    spmm = _spmm_kernel()
    aggp0 = spmm(g0, fs, fd, fw)
    hw1, g1 = _tc_mid0(aggp0, hw0, dinvf, b_s0, g_s0, be_s0, W_s1)

    aggp1 = spmm(g1, fs, fd, fw)
    hwa, ga, hwf, gf = _tc_mid1(aggp1, hw1, dinvf, dinva, b_s1, g_s1, be_s1,
                                W_a, W_f)

    aggpa = spmm(ga, asrc, adst, aw)
    aggpf = spmm(gf, fs, fd, fw)

    pa = _tc_pool(aggpa, hwa, dinva, b_a, g_a, be_a, batch2d)
    pf = _tc_pool(aggpf, hwf, dinvf, b_f, g_f, be_f, batch2d)
    return _tc_head(pa, pf, Wc1, bc1, Wc2, bc2)


# speedup vs baseline: 1.0875x; 1.0875x over previous
import functools

import jax
import jax.numpy as jnp
from jax import lax
from jax.experimental import pallas as pl
from jax.experimental.pallas import tpu as pltpu
from jax.experimental.pallas import tpu_sc as plsc

N = 10000
E = 320000
D = 128
H = 64
G = 16
OUT = 2

NC = 2
NS = 16
LN = 16
NW = NC * NS

CH = 128
NCHUNK = 80
EP = NW * NCHUNK * CH
NP = 10240
RPT = NP // NS

@functools.cache
def _sc_mesh():
    return plsc.VectorSubcoreMesh(core_axis_name="c", subcore_axis_name="s",
                                  num_cores=NC, num_subcores=NS)


def _zero_rows(rows):
    z16 = jnp.zeros((LN,), jnp.float32)

    def body(i, carry):
        r = i // (H // LN)
        q = i % (H // LN)
        rows[r, pl.ds(q * LN, LN)] = z16
        return carry

    lax.fori_loop(0, CH * (H // LN), body, 0, unroll=8)


def _deg_body(fidx_hbm, fw_hbm, aidx_hbm, aw_hbm, out_hbm,
              accf, acca, idx_v, w_v, zrow):
    cid = lax.axis_index("c")
    sid = lax.axis_index("s")
    wid = sid * NC + cid

    z16 = jnp.zeros((LN,), jnp.float32)

    def zb(i, c):
        zrow[pl.ds(i * LN, LN)] = z16
        return c

    lax.fori_loop(0, RPT // LN, zb, 0, unroll=8)
    pltpu.sync_copy(zrow, accf.at[pl.ds(sid * RPT, RPT)])
    pltpu.sync_copy(zrow, acca.at[pl.ds(sid * RPT, RPT)])
    plsc.subcore_barrier()

    def run(idx_hbm, w_hbm, acc):
        pltpu.sync_copy(idx_hbm.at[wid], idx_v)
        pltpu.sync_copy(w_hbm.at[wid], w_v)

        def body(j, c):
            pltpu.sync_copy(w_v.at[j], acc.at[idx_v.at[j]], add=True)
            return c

        lax.fori_loop(0, NCHUNK, body, 0)

    run(fidx_hbm, fw_hbm, accf)
    run(aidx_hbm, aw_hbm, acca)
    plsc.subcore_barrier()
    pltpu.sync_copy(accf.at[pl.ds(sid * RPT, RPT)],
                    out_hbm.at[cid, 0, pl.ds(sid * RPT, RPT)])
    pltpu.sync_copy(acca.at[pl.ds(sid * RPT, RPT)],
                    out_hbm.at[cid, 1, pl.ds(sid * RPT, RPT)])


@functools.cache
def _deg_kernel():
    return pl.kernel(
        _deg_body,
        out_type=jax.ShapeDtypeStruct((NC, 2, NP), jnp.float32),
        mesh=_sc_mesh(),
        compiler_params=pltpu.CompilerParams(use_tc_tiling_on_sc=False),
        scratch_types=[
            pltpu.VMEM_SHARED((NP,), jnp.float32),
            pltpu.VMEM_SHARED((NP,), jnp.float32),
            pltpu.VMEM((NCHUNK, CH), jnp.int32),
            pltpu.VMEM((NCHUNK, CH), jnp.float32),
            pltpu.VMEM((RPT,), jnp.float32),
        ],
    )


def _scale_rows(rows, w_v, j):
    for gi in range(CH // LN):
        wv = w_v[j, pl.ds(gi * LN, LN)]
        for ee in range(LN):
            wb = jnp.full((LN,), wv[ee], jnp.float32)
            e = gi * LN + ee
            for q in range(H // LN):
                rows[e, pl.ds(q * LN, LN)] = rows[e, pl.ds(q * LN, LN)] * wb


def _spmm_body(g_hbm, sidx_hbm, didx_hbm, w_hbm, out_hbm,
               acc, sidx_v, didx_v, w_v, rows0, rows1, sem0, sem1):
    cid = lax.axis_index("c")
    sid = lax.axis_index("s")
    wid = sid * NC + cid

    pltpu.sync_copy(sidx_hbm.at[wid], sidx_v)
    pltpu.sync_copy(didx_hbm.at[wid], didx_v)
    pltpu.sync_copy(w_hbm.at[wid], w_v)

    pltpu.async_copy(g_hbm.at[sidx_v.at[0]], rows0, sem0)

    _zero_rows(rows1)
    for k in range(RPT // CH):
        pltpu.sync_copy(rows1, acc.at[pl.ds(sid * RPT + k * CH, CH)])
    plsc.subcore_barrier()

    def body(jj, c):
        j0 = 2 * jj
        pltpu.async_copy(g_hbm.at[sidx_v.at[j0 + 1]], rows1, sem1)
        pltpu.make_async_copy(g_hbm.at[sidx_v.at[j0]], rows0, sem0).wait()
        _scale_rows(rows0, w_v, j0)
        pltpu.sync_copy(rows0, acc.at[didx_v.at[j0]], add=True)

        @pl.when(j0 + 2 < NCHUNK)
        def _():
            pltpu.async_copy(g_hbm.at[sidx_v.at[j0 + 2]], rows0, sem0)

        pltpu.make_async_copy(g_hbm.at[sidx_v.at[j0 + 1]], rows1, sem1).wait()
        _scale_rows(rows1, w_v, j0 + 1)
        pltpu.sync_copy(rows1, acc.at[didx_v.at[j0 + 1]], add=True)
        return c

    lax.fori_loop(0, NCHUNK // 2, body, 0)

    plsc.subcore_barrier()
    pltpu.sync_copy(acc.at[pl.ds(sid * RPT, RPT)],
                    out_hbm.at[cid, pl.ds(sid * RPT, RPT)])


@functools.cache
def _spmm_kernel():
    return pl.kernel(
        _spmm_body,
        out_type=jax.ShapeDtypeStruct((NC, NP, H), jnp.float32),
        mesh=_sc_mesh(),
        compiler_params=pltpu.CompilerParams(use_tc_tiling_on_sc=False),
        scratch_types=[
            pltpu.VMEM_SHARED((NP, H), jnp.float32),
            pltpu.VMEM((NCHUNK, CH), jnp.int32),
            pltpu.VMEM((NCHUNK, CH), jnp.int32),
            pltpu.VMEM((NCHUNK, CH), jnp.float32),
            pltpu.VMEM((CH, H), jnp.float32),
            pltpu.VMEM((CH, H), jnp.float32),
            pltpu.SemaphoreType.DMA,
            pltpu.SemaphoreType.DMA,
        ],
    )


def _bn_cols(y, g, be):
    m = jnp.mean(y, axis=0, keepdims=True)
    v = jnp.mean(y * y, axis=0, keepdims=True) - m * m
    return (y - m) * lax.rsqrt(v + 1e-5) * g[None, :] + be[None, :]


def _tc_prep_body(x_ref, w_ref, degp_ref, hw_ref, g_ref, dinvf_ref, dinva_ref):
    degf = degp_ref[0, 0, :N] + degp_ref[1, 0, :N] + 1.0
    dega = degp_ref[0, 1, :N] + degp_ref[1, 1, :N] + 1.0
    dinvf = jnp.where(degf > 0, lax.rsqrt(degf), 0.0)[:, None]
    dinva = jnp.where(dega > 0, lax.rsqrt(dega), 0.0)[:, None]
    hw = jnp.dot(x_ref[...], w_ref[...], preferred_element_type=jnp.float32)
    hw_ref[...] = hw
    g_ref[...] = dinvf * hw
    dinvf_ref[...] = dinvf
    dinva_ref[...] = dinva


_tc_prep = pl.pallas_call(
    _tc_prep_body,
    out_shape=[
        jax.ShapeDtypeStruct((N, H), jnp.float32),
        jax.ShapeDtypeStruct((N, H), jnp.float32),
        jax.ShapeDtypeStruct((N, 1), jnp.float32),
        jax.ShapeDtypeStruct((N, 1), jnp.float32),
    ],
)


def _post(aggp_ref, hw_ref, dinv_ref, b_ref, g_ref, be_ref):
    dinv = dinv_ref[...]
    agg = aggp_ref[0, :N, :] + aggp_ref[1, :N, :]
    y = dinv * agg + (dinv * dinv) * hw_ref[...] + b_ref[...][None, :]
    return _bn_cols(y, g_ref[...], be_ref[...])


def _tc_mid0_body(aggp_ref, hw_ref, dinvf_ref, b_ref, g_ref, be_ref, w1_ref,
                  hw1_ref, g1_ref):
    h0 = jax.nn.relu(_post(aggp_ref, hw_ref, dinvf_ref, b_ref, g_ref, be_ref))
    hw1 = jnp.dot(h0, w1_ref[...], preferred_element_type=jnp.float32)
    hw1_ref[...] = hw1
    g1_ref[...] = dinvf_ref[...] * hw1


_tc_mid0 = pl.pallas_call(
    _tc_mid0_body,
    out_shape=[
        jax.ShapeDtypeStruct((N, H), jnp.float32),
        jax.ShapeDtypeStruct((N, H), jnp.float32),
    ],
)


def _tc_mid1_body(aggp_ref, hw_ref, dinvf_ref, dinva_ref, b_ref, g_ref, be_ref,
                  wa_ref, wf_ref, hwa_ref, ga_ref, hwf_ref, gf_ref):
    h1 = _post(aggp_ref, hw_ref, dinvf_ref, b_ref, g_ref, be_ref)
    hwa = jnp.dot(h1, wa_ref[...], preferred_element_type=jnp.float32)
    hwa_ref[...] = hwa
    ga_ref[...] = dinva_ref[...] * hwa
    hwf = jnp.dot(h1, wf_ref[...], preferred_element_type=jnp.float32)
    hwf_ref[...] = hwf
    gf_ref[...] = dinvf_ref[...] * hwf


_tc_mid1 = pl.pallas_call(
    _tc_mid1_body,
    out_shape=[
        jax.ShapeDtypeStruct((N, H), jnp.float32),
        jax.ShapeDtypeStruct((N, H), jnp.float32),
        jax.ShapeDtypeStruct((N, H), jnp.float32),
        jax.ShapeDtypeStruct((N, H), jnp.float32),
    ],
)


def _tc_pool_body(aggp_ref, hw_ref, dinv_ref, b_ref, g_ref, be_ref, batch_ref,
                  pool_ref):
    h = _post(aggp_ref, hw_ref, dinv_ref, b_ref, g_ref, be_ref)
    oh = (batch_ref[...] == lax.broadcasted_iota(jnp.int32, (1, G), 1)
          ).astype(jnp.float32)
    cnt = jnp.maximum(jnp.sum(oh, axis=0), 1.0)[:, None]
    dn = (((0,), (0,)), ((), ()))
    pool_ref[...] = lax.dot_general(oh, h, dn,
                                    preferred_element_type=jnp.float32) / cnt


_tc_pool = pl.pallas_call(
    _tc_pool_body,
    out_shape=jax.ShapeDtypeStruct((G, H), jnp.float32),
)


def _tc_head_body(pa_ref, pf_ref, wc1_ref, bc1_ref, wc2_ref, bc2_ref, out_ref):
    combined = jnp.concatenate([pa_ref[...], pf_ref[...]], axis=1)
    z = jax.nn.relu(jnp.dot(combined, wc1_ref[...],
                            preferred_element_type=jnp.float32)
                    + bc1_ref[...][None, :])
    out_ref[...] = (jnp.dot(z, wc2_ref[...], preferred_element_type=jnp.float32)
                    + bc2_ref[...][None, :])


_tc_head = pl.pallas_call(
    _tc_head_body,
    out_shape=jax.ShapeDtypeStruct((G, OUT), jnp.float32),
)


def _prep_edges(edge_index, edge_attr):
    src = edge_index[0].astype(jnp.int32)
    dst = edge_index[1].astype(jnp.int32)
    w = edge_attr[:, 0].astype(jnp.float32)
    pad = EP - E
    pidx = (jnp.arange(pad, dtype=jnp.int32) * 37) % N
    src = jnp.concatenate([src, pidx]).reshape(NW, NCHUNK, CH)
    dst = jnp.concatenate([dst, pidx]).reshape(NW, NCHUNK, CH)
    w = jnp.concatenate([w, jnp.zeros((pad,), jnp.float32)]
                        ).reshape(NW, NCHUNK, CH)
    return src, dst, w


def kernel(x, func_edge_index, func_edge_attr, anat_edge_index, anat_edge_attr,
           batch, W_s0, b_s0, g_s0, be_s0, W_s1, b_s1, g_s1, be_s1,
           W_a, b_a, g_a, be_a, W_f, b_f, g_f, be_f, Wc1, bc1, Wc2, bc2):
    fs, fd, fw = _prep_edges(func_edge_index, func_edge_attr)
    asrc, adst, aw = _prep_edges(anat_edge_index, anat_edge_attr)
    batch2d = batch.astype(jnp.int32)[:, None]

    degp = _deg_kernel()(fd, fw, adst, aw)
    hw0, g0, dinvf, dinva = _tc_prep(x, W_s0, degp)

    spmm = _spmm_kernel()
    aggp0 = spmm(g0, fs, fd, fw)
    hw1, g1 = _tc_mid0(aggp0, hw0, dinvf, b_s0, g_s0, be_s0, W_s1)

    aggp1 = spmm(g1, fs, fd, fw)
    hwa, ga, hwf, gf = _tc_mid1(aggp1, hw1, dinvf, dinva, b_s1, g_s1, be_s1,
                                W_a, W_f)

    aggpa = spmm(ga, asrc, adst, aw)
    aggpf = spmm(gf, fs, fd, fw)

    pa = _tc_pool(aggpa, hwa, dinva, b_a, g_a, be_a, batch2d)
    pf = _tc_pool(aggpf, hwf, dinvf, b_f, g_f, be_f, batch2d)
    return _tc_head(pa, pf, Wc1, bc1, Wc2, bc2)


# speedup vs baseline: 1.1581x; 1.0649x over previous
import functools

import jax
import jax.numpy as jnp
from jax import lax
from jax.experimental import pallas as pl
from jax.experimental.pallas import tpu as pltpu
from jax.experimental.pallas import tpu_sc as plsc

N = 10000
E = 320000
D = 128
H = 64
G = 16
OUT = 2

NC = 2
NS = 16
LN = 16
NW = NC * NS

CH = 128
NCHUNK = 80
EP = NW * NCHUNK * CH
NP = 10240
RPT = NP // NS

@functools.cache
def _sc_mesh():
    return plsc.VectorSubcoreMesh(core_axis_name="c", subcore_axis_name="s",
                                  num_cores=NC, num_subcores=NS)


def _zero_rows(rows):
    z16 = jnp.zeros((LN,), jnp.float32)

    def body(i, carry):
        r = i // (H // LN)
        q = i % (H // LN)
        rows[r, pl.ds(q * LN, LN)] = z16
        return carry

    lax.fori_loop(0, CH * (H // LN), body, 0, unroll=8)


def _deg_body(fidx_hbm, fw_hbm, aidx_hbm, aw_hbm, out_hbm,
              accf, acca, idx_v, w_v, zrow):
    cid = lax.axis_index("c")
    sid = lax.axis_index("s")
    wid = sid * NC + cid

    z16 = jnp.zeros((LN,), jnp.float32)

    def zb(i, c):
        zrow[pl.ds(i * LN, LN)] = z16
        return c

    lax.fori_loop(0, RPT // LN, zb, 0, unroll=8)
    pltpu.sync_copy(zrow, accf.at[pl.ds(sid * RPT, RPT)])
    pltpu.sync_copy(zrow, acca.at[pl.ds(sid * RPT, RPT)])
    plsc.subcore_barrier()

    def run(idx_hbm, w_hbm, acc):
        pltpu.sync_copy(idx_hbm.at[wid], idx_v)
        pltpu.sync_copy(w_hbm.at[wid], w_v)

        def body(j, c):
            pltpu.sync_copy(w_v.at[j], acc.at[idx_v.at[j]], add=True)
            return c

        lax.fori_loop(0, NCHUNK, body, 0)

    run(fidx_hbm, fw_hbm, accf)
    run(aidx_hbm, aw_hbm, acca)
    plsc.subcore_barrier()
    pltpu.sync_copy(accf.at[pl.ds(sid * RPT, RPT)],
                    out_hbm.at[cid, 0, pl.ds(sid * RPT, RPT)])
    pltpu.sync_copy(acca.at[pl.ds(sid * RPT, RPT)],
                    out_hbm.at[cid, 1, pl.ds(sid * RPT, RPT)])


@functools.cache
def _deg_kernel():
    return pl.kernel(
        _deg_body,
        out_type=jax.ShapeDtypeStruct((NC, 2, NP), jnp.float32),
        mesh=_sc_mesh(),
        compiler_params=pltpu.CompilerParams(use_tc_tiling_on_sc=False),
        scratch_types=[
            pltpu.VMEM_SHARED((NP,), jnp.float32),
            pltpu.VMEM_SHARED((NP,), jnp.float32),
            pltpu.VMEM((NCHUNK, CH), jnp.int32),
            pltpu.VMEM((NCHUNK, CH), jnp.float32),
            pltpu.VMEM((RPT,), jnp.float32),
        ],
    )


def _scale_rows(rows_u, rowsf, w_v, j):
    himask = jnp.full((LN,), -65536, jnp.int32)
    sh16 = jnp.full((LN,), 16, jnp.int32)
    for gi in range(CH // LN):
        wv = w_v[j, pl.ds(gi * LN, LN)]
        for ee in range(LN):
            wb = jnp.full((LN,), wv[ee], jnp.float32)
            e = gi * LN + ee
            for t in range(H // (2 * LN)):
                vw = rows_u[e, pl.ds(t * LN, LN)]
                va = lax.bitcast_convert_type(vw << sh16, jnp.float32)
                vb = lax.bitcast_convert_type(vw & himask, jnp.float32)
                rowsf[e, pl.ds(t * LN, LN)] = va * wb
                rowsf[e, pl.ds(H // 2 + t * LN, LN)] = vb * wb


def _spmm_body(g_hbm, sidx_hbm, didx_hbm, w_hbm, out_hbm,
               acc, sidx_v, didx_v, w_v, rows0, rows1, rowsf, zbuf,
               sem0, sem1):
    cid = lax.axis_index("c")
    sid = lax.axis_index("s")
    wid = sid * NC + cid

    pltpu.sync_copy(sidx_hbm.at[wid], sidx_v)
    pltpu.sync_copy(didx_hbm.at[wid], didx_v)
    pltpu.sync_copy(w_hbm.at[wid], w_v)

    pltpu.async_copy(g_hbm.at[sidx_v.at[0]], rows0, sem0)

    _zero_rows(zbuf)
    for k in range(RPT // CH):
        pltpu.sync_copy(zbuf, acc.at[pl.ds(sid * RPT + k * CH, CH)])
    plsc.subcore_barrier()

    def body(jj, c):
        j0 = 2 * jj
        pltpu.async_copy(g_hbm.at[sidx_v.at[j0 + 1]], rows1, sem1)
        pltpu.make_async_copy(g_hbm.at[sidx_v.at[j0]], rows0, sem0).wait()
        _scale_rows(rows0, rowsf, w_v, j0)
        pltpu.sync_copy(rowsf, acc.at[didx_v.at[j0]], add=True)

        @pl.when(j0 + 2 < NCHUNK)
        def _():
            pltpu.async_copy(g_hbm.at[sidx_v.at[j0 + 2]], rows0, sem0)

        pltpu.make_async_copy(g_hbm.at[sidx_v.at[j0 + 1]], rows1, sem1).wait()
        _scale_rows(rows1, rowsf, w_v, j0 + 1)
        pltpu.sync_copy(rowsf, acc.at[didx_v.at[j0 + 1]], add=True)
        return c

    lax.fori_loop(0, NCHUNK // 2, body, 0)

    plsc.subcore_barrier()
    pltpu.sync_copy(acc.at[pl.ds(sid * RPT, RPT)],
                    out_hbm.at[cid, pl.ds(sid * RPT, RPT)])


@functools.cache
def _spmm_kernel():
    return pl.kernel(
        _spmm_body,
        out_type=jax.ShapeDtypeStruct((NC, NP, H), jnp.float32),
        mesh=_sc_mesh(),
        compiler_params=pltpu.CompilerParams(use_tc_tiling_on_sc=False),
        scratch_types=[
            pltpu.VMEM_SHARED((NP, H), jnp.float32),
            pltpu.VMEM((NCHUNK, CH), jnp.int32),
            pltpu.VMEM((NCHUNK, CH), jnp.int32),
            pltpu.VMEM((NCHUNK, CH), jnp.float32),
            pltpu.VMEM((CH, H // 2), jnp.int32),
            pltpu.VMEM((CH, H // 2), jnp.int32),
            pltpu.VMEM((CH, H), jnp.float32),
            pltpu.VMEM((CH, H), jnp.float32),
            pltpu.SemaphoreType.DMA,
            pltpu.SemaphoreType.DMA,
        ],
    )


def _pack_rows(v):
    vb = v.astype(jnp.bfloat16).astype(jnp.float32)
    bits = lax.bitcast_convert_type(vb, jnp.int32)
    lo = lax.shift_right_logical(bits[:, : H // 2], jnp.int32(16))
    hi = bits[:, H // 2:] & jnp.int32(-65536)
    return lo | hi


def _bn_cols(y, g, be):
    m = jnp.mean(y, axis=0, keepdims=True)
    v = jnp.mean(y * y, axis=0, keepdims=True) - m * m
    return (y - m) * lax.rsqrt(v + 1e-5) * g[None, :] + be[None, :]


def _tc_prep_body(x_ref, w_ref, degp_ref, hw_ref, g_ref, dinvf_ref, dinva_ref):
    degf = degp_ref[0, 0, :N] + degp_ref[1, 0, :N] + 1.0
    dega = degp_ref[0, 1, :N] + degp_ref[1, 1, :N] + 1.0
    dinvf = jnp.where(degf > 0, lax.rsqrt(degf), 0.0)[:, None]
    dinva = jnp.where(dega > 0, lax.rsqrt(dega), 0.0)[:, None]
    hw = jnp.dot(x_ref[...], w_ref[...], preferred_element_type=jnp.float32)
    hw_ref[...] = hw
    g_ref[...] = _pack_rows(dinvf * hw)
    dinvf_ref[...] = dinvf
    dinva_ref[...] = dinva


_tc_prep = pl.pallas_call(
    _tc_prep_body,
    out_shape=[
        jax.ShapeDtypeStruct((N, H), jnp.float32),
        jax.ShapeDtypeStruct((N, H // 2), jnp.int32),
        jax.ShapeDtypeStruct((N, 1), jnp.float32),
        jax.ShapeDtypeStruct((N, 1), jnp.float32),
    ],
)


def _post(aggp_ref, hw_ref, dinv_ref, b_ref, g_ref, be_ref):
    dinv = dinv_ref[...]
    agg = aggp_ref[0, :N, :] + aggp_ref[1, :N, :]
    y = dinv * agg + (dinv * dinv) * hw_ref[...] + b_ref[...][None, :]
    return _bn_cols(y, g_ref[...], be_ref[...])


def _tc_mid0_body(aggp_ref, hw_ref, dinvf_ref, b_ref, g_ref, be_ref, w1_ref,
                  hw1_ref, g1_ref):
    h0 = jax.nn.relu(_post(aggp_ref, hw_ref, dinvf_ref, b_ref, g_ref, be_ref))
    hw1 = jnp.dot(h0, w1_ref[...], preferred_element_type=jnp.float32)
    hw1_ref[...] = hw1
    g1_ref[...] = _pack_rows(dinvf_ref[...] * hw1)


_tc_mid0 = pl.pallas_call(
    _tc_mid0_body,
    out_shape=[
        jax.ShapeDtypeStruct((N, H), jnp.float32),
        jax.ShapeDtypeStruct((N, H // 2), jnp.int32),
    ],
)


def _tc_mid1_body(aggp_ref, hw_ref, dinvf_ref, dinva_ref, b_ref, g_ref, be_ref,
                  wa_ref, wf_ref, hwa_ref, ga_ref, hwf_ref, gf_ref):
    h1 = _post(aggp_ref, hw_ref, dinvf_ref, b_ref, g_ref, be_ref)
    hwa = jnp.dot(h1, wa_ref[...], preferred_element_type=jnp.float32)
    hwa_ref[...] = hwa
    ga_ref[...] = _pack_rows(dinva_ref[...] * hwa)
    hwf = jnp.dot(h1, wf_ref[...], preferred_element_type=jnp.float32)
    hwf_ref[...] = hwf
    gf_ref[...] = _pack_rows(dinvf_ref[...] * hwf)


_tc_mid1 = pl.pallas_call(
    _tc_mid1_body,
    out_shape=[
        jax.ShapeDtypeStruct((N, H), jnp.float32),
        jax.ShapeDtypeStruct((N, H // 2), jnp.int32),
        jax.ShapeDtypeStruct((N, H), jnp.float32),
        jax.ShapeDtypeStruct((N, H // 2), jnp.int32),
    ],
)


def _tc_pool_body(aggp_ref, hw_ref, dinv_ref, b_ref, g_ref, be_ref, batch_ref,
                  pool_ref):
    h = _post(aggp_ref, hw_ref, dinv_ref, b_ref, g_ref, be_ref)
    oh = (batch_ref[...] == lax.broadcasted_iota(jnp.int32, (1, G), 1)
          ).astype(jnp.float32)
    cnt = jnp.maximum(jnp.sum(oh, axis=0), 1.0)[:, None]
    dn = (((0,), (0,)), ((), ()))
    pool_ref[...] = lax.dot_general(oh, h, dn,
                                    preferred_element_type=jnp.float32) / cnt


_tc_pool = pl.pallas_call(
    _tc_pool_body,
    out_shape=jax.ShapeDtypeStruct((G, H), jnp.float32),
)


def _tc_head_body(pa_ref, pf_ref, wc1_ref, bc1_ref, wc2_ref, bc2_ref, out_ref):
    combined = jnp.concatenate([pa_ref[...], pf_ref[...]], axis=1)
    z = jax.nn.relu(jnp.dot(combined, wc1_ref[...],
                            preferred_element_type=jnp.float32)
                    + bc1_ref[...][None, :])
    out_ref[...] = (jnp.dot(z, wc2_ref[...], preferred_element_type=jnp.float32)
                    + bc2_ref[...][None, :])


_tc_head = pl.pallas_call(
    _tc_head_body,
    out_shape=jax.ShapeDtypeStruct((G, OUT), jnp.float32),
)


def _prep_edges(edge_index, edge_attr):
    src = edge_index[0].astype(jnp.int32)
    dst = edge_index[1].astype(jnp.int32)
    w = edge_attr[:, 0].astype(jnp.float32)
    pad = EP - E
    pidx = (jnp.arange(pad, dtype=jnp.int32) * 37) % N
    src = jnp.concatenate([src, pidx]).reshape(NW, NCHUNK, CH)
    dst = jnp.concatenate([dst, pidx]).reshape(NW, NCHUNK, CH)
    w = jnp.concatenate([w, jnp.zeros((pad,), jnp.float32)]
                        ).reshape(NW, NCHUNK, CH)
    return src, dst, w


def kernel(x, func_edge_index, func_edge_attr, anat_edge_index, anat_edge_attr,
           batch, W_s0, b_s0, g_s0, be_s0, W_s1, b_s1, g_s1, be_s1,
           W_a, b_a, g_a, be_a, W_f, b_f, g_f, be_f, Wc1, bc1, Wc2, bc2):
    fs, fd, fw = _prep_edges(func_edge_index, func_edge_attr)
    asrc, adst, aw = _prep_edges(anat_edge_index, anat_edge_attr)
    batch2d = batch.astype(jnp.int32)[:, None]

    degp = _deg_kernel()(fd, fw, adst, aw)
    hw0, g0, dinvf, dinva = _tc_prep(x, W_s0, degp)

    spmm = _spmm_kernel()
    aggp0 = spmm(g0, fs, fd, fw)
    hw1, g1 = _tc_mid0(aggp0, hw0, dinvf, b_s0, g_s0, be_s0, W_s1)

    aggp1 = spmm(g1, fs, fd, fw)
    hwa, ga, hwf, gf = _tc_mid1(aggp1, hw1, dinvf, dinva, b_s1, g_s1, be_s1,
                                W_a, W_f)

    aggpa = spmm(ga, asrc, adst, aw)
    aggpf = spmm(gf, fs, fd, fw)

    pa = _tc_pool(aggpa, hwa, dinva, b_a, g_a, be_a, batch2d)
    pf = _tc_pool(aggpf, hwf, dinvf, b_f, g_f, be_f, batch2d)
    return _tc_head(pa, pf, Wc1, bc1, Wc2, bc2)
